# baseline (speedup 1.0000x reference)
import functools

import jax
import jax.numpy as jnp
from jax.experimental import pallas as pl
from jax.experimental.pallas import tpu as pltpu

EMBED_DIM = 1024
NUM_HEADS = 16
NUM_POINTS = 4
HEAD_DIM = EMBED_DIM // NUM_HEADS


def _matmul_bias_kernel(x_ref, w_ref, b_ref, o_ref):
    o_ref[...] = (
        jnp.dot(x_ref[...], w_ref[...], preferred_element_type=jnp.float32)
        + b_ref[...]
    )


def _matmul_bias(x, w_t, b, bm=512):
    M, K = x.shape
    N = w_t.shape[1]
    grid = (M // bm,)
    return pl.pallas_call(
        _matmul_bias_kernel,
        grid=grid,
        in_specs=[
            pl.BlockSpec((bm, K), lambda i: (i, 0)),
            pl.BlockSpec((K, N), lambda i: (0, 0)),
            pl.BlockSpec((1, N), lambda i: (0, 0)),
        ],
        out_specs=pl.BlockSpec((bm, N), lambda i: (i, 0)),
        out_shape=jax.ShapeDtypeStruct((M, N), jnp.float32),
    )(x, w_t, b.reshape(1, N))


def kernel(query, value, value_key_padding_mask, value_valid_ratio,
           reference_point, snippet_num, W_so, b_so, W_aw, b_aw, W_v, b_v,
           W_o, b_o):
    Lq, bz, d = query.shape
    Lv = value.shape[0]
    H, P, hd = NUM_HEADS, NUM_POINTS, HEAD_DIM

    v = _matmul_bias(value.reshape(Lv * bz, d), W_v.T, b_v).reshape(Lv, bz, d)
    v = jnp.where(value_key_padding_mask.T[:, :, None], 0.0, v)
    v = jnp.transpose(v, (1, 2, 0)).reshape(bz * H, hd, Lv)

    qf = query.reshape(Lq * bz, d)
    offset = _matmul_bias(qf, W_so.T, b_so).reshape(Lq, bz, H, P)
    offset = jnp.transpose(offset, (1, 2, 0, 3))
    ref_c = reference_point[..., :1].reshape(bz, 1, Lq, 1)
    ref_w = reference_point[..., 1:].reshape(bz, 1, Lq, 1)
    offset = ref_c + offset / P * ref_w * 0.5
    offset = offset * 2.0 - 1.0
    offset = offset.reshape(bz * H, Lq * P)

    aw = _matmul_bias(qf, W_aw.T, b_aw).reshape(Lq, bz, H, P)
    aw = jnp.transpose(aw, (1, 2, 0, 3))
    aw = jax.nn.softmax(aw, axis=-1).reshape(bz * H * Lq, P, 1)

    x = (offset + 1.0) * 0.5 * (Lv - 1)
    x0f = jnp.floor(x)
    x0 = x0f.astype(jnp.int32)
    x1 = x0 + 1
    w1 = x - x0f
    w0 = 1.0 - w1

    def gath(vv, idx):
        return vv[:, jnp.clip(idx, 0, Lv - 1)]

    v0 = jax.vmap(gath)(v, x0)
    v1 = jax.vmap(gath)(v, x1)
    m0 = ((x0 >= 0) & (x0 <= Lv - 1)).astype(v.dtype)
    m1 = ((x1 >= 0) & (x1 <= Lv - 1)).astype(v.dtype)
    sampled = w0[:, None, :] * v0 * m0[:, None, :] + w1[:, None, :] * v1 * m1[:, None, :]
    sampled = sampled.reshape(bz * H, hd, Lq, P)
    sampled = jnp.transpose(sampled, (0, 2, 1, 3)).reshape(bz * H * Lq, hd, P)
    attn_out = jnp.matmul(sampled, aw)
    attn_out = attn_out.reshape(bz, H, Lq, hd)
    attn_out = jnp.transpose(attn_out, (2, 0, 1, 3)).reshape(Lq, bz, d)

    out = _matmul_bias(attn_out.reshape(Lq * bz, d), W_o.T, b_o)
    return out.reshape(Lq, bz, d)


# speedup vs baseline: 20.6327x; 20.6327x over previous
import functools

import jax
import jax.numpy as jnp
from jax import lax
from jax.experimental import pallas as pl
from jax.experimental.pallas import tpu as pltpu
from jax.experimental.pallas import tpu_sc as plsc

EMBED_DIM = 1024
NUM_HEADS = 16
NUM_POINTS = 4
HEAD_DIM = EMBED_DIM // NUM_HEADS
LQ = 2048
LV = 2048
BZ = 4

NC, NS, L = 2, 16, 16
NW = NC * NS

R_TOTAL = LQ * BZ * NUM_HEADS
ROWB = LQ * BZ
ENTRIES = ROWB * 128
RB_PER_CHUNK = 4
CHUNK_E = RB_PER_CHUNK * 128
CHUNK_R = RB_PER_CHUNK * NUM_HEADS
RB_PER_W = ROWB // NW
CHUNKS_PER_W = RB_PER_W // RB_PER_CHUNK


def _matmul_bias_kernel(x_ref, w_ref, b_ref, o_ref):
    o_ref[...] = (
        jnp.dot(x_ref[...], w_ref[...], preferred_element_type=jnp.float32)
        + b_ref[...]
    )


def _matmul_bias_mask_kernel(x_ref, w_ref, b_ref, m_ref, o_ref):
    o_ref[...] = (
        jnp.dot(x_ref[...], w_ref[...], preferred_element_type=jnp.float32)
        + b_ref[...]
    ) * m_ref[...]


def _matmul_bias(x, w_t, b, mask_col=None, bm=512):
    M, K = x.shape
    N = w_t.shape[1]
    grid = (M // bm,)
    if mask_col is None:
        return pl.pallas_call(
            _matmul_bias_kernel,
            grid=grid,
            in_specs=[
                pl.BlockSpec((bm, K), lambda i: (i, 0)),
                pl.BlockSpec((K, N), lambda i: (0, 0)),
                pl.BlockSpec((1, N), lambda i: (0, 0)),
            ],
            out_specs=pl.BlockSpec((bm, N), lambda i: (i, 0)),
            out_shape=jax.ShapeDtypeStruct((M, N), jnp.float32),
        )(x, w_t, b.reshape(1, N))
    return pl.pallas_call(
        _matmul_bias_mask_kernel,
        grid=grid,
        in_specs=[
            pl.BlockSpec((bm, K), lambda i: (i, 0)),
            pl.BlockSpec((K, N), lambda i: (0, 0)),
            pl.BlockSpec((1, N), lambda i: (0, 0)),
            pl.BlockSpec((bm, 1), lambda i: (i, 0)),
        ],
        out_specs=pl.BlockSpec((bm, N), lambda i: (i, 0)),
        out_shape=jax.ShapeDtypeStruct((M, N), jnp.float32),
    )(x, w_t, b.reshape(1, N), mask_col)


def _prep_kernel(bm, q_ref, w_ref, b_ref, rp_ref, idx_ref, coeff_ref):
    soaw = (
        jnp.dot(q_ref[...], w_ref[...], preferred_element_type=jnp.float32)
        + b_ref[...]
    )
    rp = rp_ref[...]
    ref_c = rp[:, 0:1]
    ref_w = rp[:, 1:2]
    i = pl.program_id(0)
    row = lax.broadcasted_iota(jnp.int32, (bm, 1), 0) + i * bm
    boff = (row % BZ) * NUM_HEADS
    h_iota = lax.broadcasted_iota(jnp.int32, (bm, NUM_HEADS), 1)

    a = [soaw[:, 64 + p * 16:64 + (p + 1) * 16] for p in range(4)]
    mx = jnp.maximum(jnp.maximum(a[0], a[1]), jnp.maximum(a[2], a[3]))
    e = [jnp.exp(x - mx) for x in a]
    inv = 1.0 / (e[0] + e[1] + e[2] + e[3])

    for p in range(4):
        x = (ref_c + soaw[:, p * 16:(p + 1) * 16] * (ref_w * 0.125)) * float(LV - 1)
        x0f = jnp.floor(x)
        w1 = x - x0f
        w0 = 1.0 - w1
        x0 = x0f.astype(jnp.int32)
        x1 = x0 + 1
        m0 = (x0 >= 0) & (x0 <= LV - 1)
        m1 = (x1 >= 0) & (x1 <= LV - 1)
        awp = e[p] * inv
        swap = x0 == -1
        c_lo = jnp.where(swap, awp * w1, jnp.where(m0, awp * w0, 0.0))
        c_hi = jnp.where(swap, 0.0, jnp.where(m1, awp * w1, 0.0))
        lv0 = jnp.clip(x0, 0, LV - 1)
        idx_ref[:, p * 16:(p + 1) * 16] = lv0 * (BZ * NUM_HEADS) + boff + h_iota
        coeff_ref[:, (2 * p) * 16:(2 * p + 1) * 16] = c_lo
        coeff_ref[:, (2 * p + 1) * 16:(2 * p + 2) * 16] = c_hi


def _prep(qf, w_cat, b_cat, refq, bm=512):
    M = qf.shape[0]
    grid = (M // bm,)
    return pl.pallas_call(
        functools.partial(_prep_kernel, bm),
        grid=grid,
        in_specs=[
            pl.BlockSpec((bm, EMBED_DIM), lambda i: (i, 0)),
            pl.BlockSpec((EMBED_DIM, 128), lambda i: (0, 0)),
            pl.BlockSpec((1, 128), lambda i: (0, 0)),
            pl.BlockSpec((bm, 2), lambda i: (i, 0)),
        ],
        out_specs=[
            pl.BlockSpec((bm, 64), lambda i: (i, 0)),
            pl.BlockSpec((bm, 128), lambda i: (i, 0)),
        ],
        out_shape=[
            jax.ShapeDtypeStruct((M, 64), jnp.int32),
            jax.ShapeDtypeStruct((M, 128), jnp.float32),
        ],
    )(qf, w_cat, b_cat.reshape(1, 128), refq)


def _sc_gather_kernel(table_hbm, idx_hbm, coeff_hbm, out_hbm,
                      idx_v, coeff_v, rows_v, out_v, sem):
    wid = lax.axis_index("s") * NC + lax.axis_index("c")

    def splat(vec, h_full):
        return lax.gather(
            vec, h_full[:, None],
            lax.GatherDimensionNumbers(
                offset_dims=(), collapsed_slice_dims=(0,),
                start_index_map=(0,)),
            slice_sizes=(1,),
            mode=lax.GatherScatterMode.PROMISE_IN_BOUNDS)

    def chunk_body(g, carry):
        rb0 = wid * RB_PER_W + g * RB_PER_CHUNK
        pltpu.sync_copy(idx_hbm.at[pl.ds(rb0, RB_PER_CHUNK), :], idx_v)
        pltpu.sync_copy(coeff_hbm.at[pl.ds(rb0 * 128, CHUNK_E)], coeff_v)
        descs = []
        for i in range(RB_PER_CHUNK):
            descs.append(pltpu.async_copy(
                table_hbm.at[idx_v.at[i]],
                rows_v.at[pl.ds(i * 64, 64), :],
                sem,
            ))
        for dsc in descs:
            dsc.wait()

        def row_body(r, carry2):
            rb = r // NUM_HEADS
            h = r % NUM_HEADS
            h_full = jnp.full((L,), h, jnp.int32)
            acc = [jnp.zeros((L,), jnp.float32) for _ in range(4)]
            for p in range(4):
                pos = rb * 64 + p * 16 + h
                c_lo = splat(coeff_v[pl.ds(rb * 128 + p * 32, L)], h_full)
                c_hi = splat(coeff_v[pl.ds(rb * 128 + p * 32 + 16, L)], h_full)
                for j in range(4):
                    acc[j] = (acc[j]
                              + c_lo * rows_v[pos, pl.ds(j * 16, L)]
                              + c_hi * rows_v[pos, pl.ds(64 + j * 16, L)])
            for j in range(4):
                out_v[r, pl.ds(j * 16, L)] = acc[j]
            return carry2

        lax.fori_loop(0, CHUNK_R, row_body, 0, unroll=2)
        pltpu.sync_copy(out_v, out_hbm.at[pl.ds(rb0 * NUM_HEADS, CHUNK_R), :])
        return carry

    lax.fori_loop(0, CHUNKS_PER_W, chunk_body, 0)


@functools.cache
def _sc_gather_fn():
    return pl.kernel(
        _sc_gather_kernel,
        out_type=jax.ShapeDtypeStruct((R_TOTAL, HEAD_DIM), jnp.float32),
        mesh=plsc.VectorSubcoreMesh(core_axis_name="c", subcore_axis_name="s",
                                    num_cores=NC, num_subcores=NS),
        scratch_types=[
            pltpu.VMEM((RB_PER_CHUNK, 64), jnp.int32),
            pltpu.VMEM((CHUNK_E,), jnp.float32),
            pltpu.VMEM((RB_PER_CHUNK * 64, 2 * HEAD_DIM), jnp.float32),
            pltpu.VMEM((CHUNK_R, HEAD_DIM), jnp.float32),
            pltpu.SemaphoreType.DMA,
        ],
    )


def _sc_gather(table, idx_all, coeff_all):
    return _sc_gather_fn()(table, idx_all, coeff_all.reshape(-1))


_PERM = [ (c % 16) * 4 + c // 16 for c in range(64) ]


def kernel(query, value, value_key_padding_mask, value_valid_ratio,
           reference_point, snippet_num, W_so, b_so, W_aw, b_aw, W_v, b_v,
           W_o, b_o):
    Lq, bz, d = query.shape
    Lv = value.shape[0]

    perm = jnp.array(_PERM, dtype=jnp.int32)
    w_cat = jnp.concatenate([W_so.T[:, perm], W_aw.T[:, perm]], axis=1)
    b_cat = jnp.concatenate([b_so[perm], b_aw[perm]])

    maskf = 1.0 - value_key_padding_mask.T.reshape(Lv * bz, 1).astype(jnp.float32)
    v3 = _matmul_bias(value.reshape(Lv * bz, d), W_v.T, b_v,
                      mask_col=maskf).reshape(Lv, bz * NUM_HEADS, HEAD_DIM)
    nxt = jnp.concatenate(
        [v3[1:], jnp.zeros((1, bz * NUM_HEADS, HEAD_DIM), jnp.float32)], axis=0)
    table = jnp.concatenate([v3, nxt], axis=-1).reshape(R_TOTAL, 2 * HEAD_DIM)

    refq = jnp.transpose(reference_point, (1, 0, 2)).reshape(Lq * bz, 2)
    qf = query.reshape(Lq * bz, d)
    idx_all, coeff_all = _prep(qf, w_cat, b_cat, refq)

    attn = _sc_gather(table, idx_all, coeff_all)

    out = _matmul_bias(attn.reshape(Lq * bz, d), W_o.T, b_o)
    return out.reshape(Lq, bz, d)
